# baseline (speedup 1.0000x reference)
import functools

import numpy as np
import jax
import jax.numpy as jnp
from jax import lax
from jax.experimental import pallas as pl
from jax.experimental.pallas import tpu as pltpu
from jax.experimental.pallas import tpu_sc as plsc

_MARGIN = 0.1
_N_PAIRS = 100000
_LANES = 16
_NC = 2
_NS = 16
_NW = _NC * _NS

_idx_cache = {}


def _threefry2x32(k1, k2, x1, x2):
    def rotl(x, r):
        return ((x << np.uint32(r)) | (x >> np.uint32(32 - r))).astype(np.uint32)

    x1 = x1.astype(np.uint32).copy()
    x2 = x2.astype(np.uint32).copy()
    ks0 = np.uint32(k1)
    ks1 = np.uint32(k2)
    ks2 = np.uint32(ks0 ^ ks1 ^ np.uint32(0x1BD11BDA))
    ks = [ks0, ks1, ks2]
    rot = [(13, 15, 26, 6), (17, 29, 16, 24)]
    x1 += ks0
    x2 += ks1
    for i in range(5):
        for r in rot[i % 2]:
            x1 += x2
            x2 = rotl(x2, r)
            x2 ^= x1
        x1 += ks[(i + 1) % 3]
        x2 += ks[(i + 2) % 3] + np.uint32(i + 1)
    return x1, x2


def _tf_split(kd, num):
    w0, w1 = _threefry2x32(kd[0], kd[1], np.zeros(num, np.uint32),
                           np.arange(num, dtype=np.uint32))
    return np.stack([w0, w1], axis=1)


def _tf_bits32(kd, n):
    w0, w1 = _threefry2x32(kd[0], kd[1], np.zeros(n, np.uint32),
                           np.arange(n, dtype=np.uint32))
    return w0 ^ w1


def _tf_randint(kd, n, minval, maxval):
    ks = _tf_split(kd, 2)
    higher = _tf_bits32(ks[0], n)
    lower = _tf_bits32(ks[1], n)
    span = np.uint32(maxval - minval)
    mult = np.uint32((np.uint64(65536 % int(span)) ** 2) % np.uint64(span))
    out = ((higher % span) * mult + lower % span) % span
    return np.int32(minval) + out.astype(np.int32)


def _pair_indices(batch_size):
    got = _idx_cache.get(batch_size)
    if got is not None:
        return got
    n_pairs = min(_N_PAIRS, batch_size * (batch_size - 1) // 2)
    root = np.array([0, 42], dtype=np.uint32)
    ks = _tf_split(root, 2)
    idx_i = _tf_randint(ks[0], n_pairs, 0, batch_size)
    idx_j = _tf_randint(ks[1], n_pairs, 0, batch_size)
    keep = idx_i != idx_j
    idx_i, idx_j = idx_i[keep], idx_j[keep]
    order = np.argsort(idx_i, kind="stable")
    idx_i, idx_j = idx_i[order], idx_j[order]
    n_eff = idx_i.shape[0]
    shift = max(1, (batch_size - 1).bit_length())
    chunk = 4 * _LANES
    per_tile = -(-n_eff // (_NW * chunk)) * chunk
    n_pad = per_tile * _NW
    packed = np.zeros(n_pad, np.int32)
    packed[:n_eff] = (idx_i.astype(np.int64) << shift | idx_j).astype(np.int32)
    got = (jnp.asarray(packed), per_tile, shift)
    _idx_cache[batch_size] = got
    return got


@functools.lru_cache(maxsize=None)
def _make_sc_call(batch, per_tile, shift):
    mask_lo = jnp.int32((1 << shift) - 1)
    mesh = plsc.VectorSubcoreMesh(core_axis_name="c", subcore_axis_name="s")

    @functools.partial(
        pl.kernel,
        mesh=mesh,
        compiler_params=pltpu.CompilerParams(needs_layout_passes=False),
        out_type=jax.ShapeDtypeStruct((2, _NW, _LANES), jnp.float32),
        scratch_types=[
            pltpu.VMEM((batch,), jnp.float32),
            pltpu.VMEM((batch,), jnp.float32),
            pltpu.VMEM((per_tile,), jnp.int32),
            pltpu.VMEM((_LANES,), jnp.float32),
            pltpu.VMEM((_LANES,), jnp.float32),
            pltpu.SemaphoreType.DMA,
        ],
    )
    def call(pred_hbm, targ_hbm, pk_hbm, out_hbm,
             pred_v, targ_v, pk_v, lo_v, cn_v, sem):
        wid = lax.axis_index("s") * _NC + lax.axis_index("c")
        base = wid * per_tile
        cps = [
            pltpu.async_copy(pred_hbm, pred_v, sem),
            pltpu.async_copy(targ_hbm, targ_v, sem),
            pltpu.async_copy(pk_hbm.at[pl.ds(base, per_tile)], pk_v, sem),
        ]
        for cp in cps:
            cp.wait()

        zero = jnp.zeros((_LANES,), jnp.float32)
        width = 4

        @plsc.parallel_loop(0, per_tile, width * _LANES, unroll=2,
                            carry=(zero,) * (2 * width))
        def acc(off, carry):
            out = []
            for k in range(width):
                al, ac = carry[2 * k], carry[2 * k + 1]
                pk = pk_v[pl.ds(off + k * _LANES, _LANES)]
                ii = lax.shift_right_logical(pk, jnp.int32(shift))
                jj = pk & mask_lo
                pi = plsc.load_gather(pred_v, [ii])
                pj = plsc.load_gather(pred_v, [jj])
                ti = plsc.load_gather(targ_v, [ii])
                tj = plsc.load_gather(targ_v, [jj])
                d = pi - pj
                sd = jnp.where(ti > tj, d, -d)
                loss = jnp.maximum(_MARGIN - sd, jnp.float32(0.0))
                m = ti != tj
                out.append(al + jnp.where(m, loss, jnp.float32(0.0)))
                out.append(ac + jnp.where(m, jnp.float32(1.0), jnp.float32(0.0)))
            return tuple(out)

        al = acc[0] + acc[2] + acc[4] + acc[6]
        ac = acc[1] + acc[3] + acc[5] + acc[7]
        lo_v[...] = al
        cn_v[...] = ac
        pltpu.sync_copy(lo_v, out_hbm.at[0, wid])
        pltpu.sync_copy(cn_v, out_hbm.at[1, wid])

    return call


def kernel(predictions, targets):
    batch = predictions.shape[0]
    packed, per_tile, shift = _pair_indices(batch)
    call = _make_sc_call(batch, per_tile, shift)
    parts = call(predictions, targets, packed)
    total = parts[0].sum()
    cnt = parts[1].sum()
    return total / jnp.maximum(cnt, 1.0)


# speedup vs baseline: 1.0917x; 1.0917x over previous
import functools

import numpy as np
import jax
import jax.numpy as jnp
from jax import lax
from jax.experimental import pallas as pl
from jax.experimental.pallas import tpu as pltpu
from jax.experimental.pallas import tpu_sc as plsc

_MARGIN = 0.1
_N_PAIRS = 100000
_LANES = 16
_NC = 2
_NS = 16
_NW = _NC * _NS

_idx_cache = {}


def _threefry2x32(k1, k2, x1, x2):
    def rotl(x, r):
        return ((x << np.uint32(r)) | (x >> np.uint32(32 - r))).astype(np.uint32)

    x1 = x1.astype(np.uint32).copy()
    x2 = x2.astype(np.uint32).copy()
    ks0 = np.uint32(k1)
    ks1 = np.uint32(k2)
    ks2 = np.uint32(ks0 ^ ks1 ^ np.uint32(0x1BD11BDA))
    ks = [ks0, ks1, ks2]
    rot = [(13, 15, 26, 6), (17, 29, 16, 24)]
    x1 += ks0
    x2 += ks1
    for i in range(5):
        for r in rot[i % 2]:
            x1 += x2
            x2 = rotl(x2, r)
            x2 ^= x1
        x1 += ks[(i + 1) % 3]
        x2 += ks[(i + 2) % 3] + np.uint32(i + 1)
    return x1, x2


def _tf_split(kd, num):
    w0, w1 = _threefry2x32(kd[0], kd[1], np.zeros(num, np.uint32),
                           np.arange(num, dtype=np.uint32))
    return np.stack([w0, w1], axis=1)


def _tf_bits32(kd, n):
    w0, w1 = _threefry2x32(kd[0], kd[1], np.zeros(n, np.uint32),
                           np.arange(n, dtype=np.uint32))
    return w0 ^ w1


def _tf_randint(kd, n, minval, maxval):
    ks = _tf_split(kd, 2)
    higher = _tf_bits32(ks[0], n)
    lower = _tf_bits32(ks[1], n)
    span = np.uint32(maxval - minval)
    mult = np.uint32((np.uint64(65536 % int(span)) ** 2) % np.uint64(span))
    out = ((higher % span) * mult + lower % span) % span
    return np.int32(minval) + out.astype(np.int32)


def _pair_indices(batch_size):
    got = _idx_cache.get(batch_size)
    if got is not None:
        return got
    n_pairs = min(_N_PAIRS, batch_size * (batch_size - 1) // 2)
    root = np.array([0, 42], dtype=np.uint32)
    ks = _tf_split(root, 2)
    idx_i = _tf_randint(ks[0], n_pairs, 0, batch_size)
    idx_j = _tf_randint(ks[1], n_pairs, 0, batch_size)
    keep = idx_i != idx_j
    idx_i, idx_j = idx_i[keep], idx_j[keep]
    n_eff = idx_i.shape[0]
    chunk = 4 * _LANES
    if batch_size % 32 == 0:
        ni, nj = 4, 8
    else:
        ni, nj = 1, 1
    si, sj = batch_size // ni, batch_size // nj
    shift = max(1, (sj - 1).bit_length())
    bucket = (idx_i // si) * nj + (idx_j // sj)
    order = np.argsort(bucket * np.int64(batch_size) + idx_i, kind="stable")
    idx_i, idx_j, bucket = idx_i[order], idx_j[order], bucket[order]
    counts = np.bincount(bucket, minlength=_NW)
    per_tile = -(-int(counts.max()) // chunk) * chunk
    packed = np.full(per_tile * _NW, -1, np.int32)
    vals = ((idx_i % si).astype(np.int64) << shift | (idx_j % sj)).astype(np.int32)
    starts = np.concatenate([[0], np.cumsum(counts)[:-1]])
    for b in range(_NW):
        seg = vals[starts[b]:starts[b] + counts[b]]
        packed[b * per_tile:b * per_tile + counts[b]] = seg
    got = (jnp.asarray(packed), per_tile, shift, si, sj)
    _idx_cache[batch_size] = got
    return got


@functools.lru_cache(maxsize=None)
def _make_sc_call(batch, per_tile, shift, si, sj):
    si_pow2 = si & (si - 1) == 0
    mask_i = jnp.int32(si - 1) if si_pow2 else jnp.int32((1 << si.bit_length()) - 1)
    mask_lo = jnp.int32((1 << shift) - 1)
    nj = batch // sj
    mesh = plsc.VectorSubcoreMesh(core_axis_name="c", subcore_axis_name="s")

    @functools.partial(
        pl.kernel,
        mesh=mesh,
        compiler_params=pltpu.CompilerParams(needs_layout_passes=False),
        out_type=jax.ShapeDtypeStruct((2, _NW, _LANES), jnp.float32),
        scratch_types=[
            pltpu.VMEM((si,), jnp.float32),
            pltpu.VMEM((si,), jnp.float32),
            pltpu.VMEM((sj,), jnp.float32),
            pltpu.VMEM((sj,), jnp.float32),
            pltpu.VMEM((per_tile,), jnp.int32),
            pltpu.VMEM((_LANES,), jnp.float32),
            pltpu.VMEM((_LANES,), jnp.float32),
            pltpu.SemaphoreType.DMA,
        ],
    )
    def call(pred_hbm, targ_hbm, pk_hbm, out_hbm,
             predi_v, targi_v, predj_v, targj_v, pk_v, lo_v, cn_v, sem):
        wid = lax.axis_index("s") * _NC + lax.axis_index("c")
        ilo = (wid // nj) * si
        jlo = (wid % nj) * sj
        base = wid * per_tile
        cps = [
            pltpu.async_copy(pk_hbm.at[pl.ds(base, per_tile)], pk_v, sem),
            pltpu.async_copy(pred_hbm.at[pl.ds(ilo, si)], predi_v, sem),
            pltpu.async_copy(targ_hbm.at[pl.ds(ilo, si)], targi_v, sem),
            pltpu.async_copy(pred_hbm.at[pl.ds(jlo, sj)], predj_v, sem),
            pltpu.async_copy(targ_hbm.at[pl.ds(jlo, sj)], targj_v, sem),
        ]
        for cp in cps:
            cp.wait()

        zero = jnp.zeros((_LANES,), jnp.float32)
        width = 4

        @plsc.parallel_loop(0, per_tile, width * _LANES, unroll=2,
                            carry=(zero,) * (2 * width))
        def acc(off, carry):
            out = []
            for k in range(width):
                al, ac = carry[2 * k], carry[2 * k + 1]
                pk = pk_v[pl.ds(off + k * _LANES, _LANES)]
                ii = lax.shift_right_logical(pk, jnp.int32(shift)) & mask_i
                jj = pk & mask_lo
                if not si_pow2:
                    ii = jnp.minimum(ii, jnp.int32(si - 1))
                    jj = jnp.minimum(jj, jnp.int32(sj - 1))
                pi = plsc.load_gather(predi_v, [ii])
                pj = plsc.load_gather(predj_v, [jj])
                ti = plsc.load_gather(targi_v, [ii])
                tj = plsc.load_gather(targj_v, [jj])
                d = pi - pj
                sd = jnp.where(ti > tj, d, -d)
                loss = jnp.maximum(_MARGIN - sd, jnp.float32(0.0))
                m = (ti != tj) & (pk >= 0)
                out.append(al + jnp.where(m, loss, jnp.float32(0.0)))
                out.append(ac + jnp.where(m, jnp.float32(1.0), jnp.float32(0.0)))
            return tuple(out)

        al = acc[0] + acc[2] + acc[4] + acc[6]
        ac = acc[1] + acc[3] + acc[5] + acc[7]
        lo_v[...] = al
        cn_v[...] = ac
        pltpu.sync_copy(lo_v, out_hbm.at[0, wid])
        pltpu.sync_copy(cn_v, out_hbm.at[1, wid])

    return call


def kernel(predictions, targets):
    batch = predictions.shape[0]
    packed, per_tile, shift, si, sj = _pair_indices(batch)
    call = _make_sc_call(batch, per_tile, shift, si, sj)
    parts = call(predictions, targets, packed)
    total = parts[0].sum()
    cnt = parts[1].sum()
    return total / jnp.maximum(cnt, 1.0)
